# baseline (speedup 1.0000x reference)
import functools

import jax
import jax.numpy as jnp
from jax import lax
from jax.experimental import pallas as pl
from jax.experimental.pallas import tpu as pltpu
from jax.experimental.pallas import tpu_sc as plsc

D_MODEL = 1024
BATCH = 4
SEQ = 2048
ROWS = BATCH * SEQ
NUM_CORES = 2
NUM_SUBCORES = 16
NW = NUM_CORES * NUM_SUBCORES
RPW = ROWS // NW
CHUNK = 32
NCHUNK = RPW // CHUNK
LANES = 16
W_PER_BATCH = NW // BATCH


def _sc_body(x_hbm, ids_hbm, table_hbm, out_hbm, ids_v, idx_v, x_v, rows_v, sem):
  cid = lax.axis_index("c")
  sid = lax.axis_index("s")
  wid = sid * NUM_CORES + cid
  b = wid // W_PER_BATCH
  s_local = (wid % W_PER_BATCH) * RPW

  pltpu.sync_copy(ids_hbm.at[pl.ds(b * SEQ, SEQ)], ids_v)

  def _mask(v):
    return jnp.where(
        v != jnp.zeros((LANES,), jnp.int32),
        jnp.ones((LANES,), jnp.int32),
        jnp.zeros((LANES,), jnp.int32),
    )

  def pre_body(i, carry):
    m = _mask(ids_v[pl.ds(i * LANES, LANES)])
    return carry + jnp.sum(m)

  carry = lax.fori_loop(0, s_local // LANES, pre_body, jnp.int32(0))

  def pos_body(j, carry):
    m = _mask(ids_v[pl.ds(s_local + j * LANES, LANES)])
    pos = (plsc.cumsum(m) + carry) * m
    idx_v[pl.ds(j * LANES, LANES)] = pos
    return carry + jnp.sum(m)

  lax.fori_loop(0, RPW // LANES, pos_body, carry)

  def chunk_body(c, _):
    r0 = wid * RPW + c * CHUNK
    gather = pltpu.async_copy(
        table_hbm.at[idx_v.at[pl.ds(c * CHUNK, CHUNK)]], rows_v, sem
    )
    pltpu.sync_copy(x_hbm.at[pl.ds(r0, CHUNK)], x_v)
    gather.wait()

    def add_row(r, _):
      def add_vec(v, _):
        sl = pl.ds(v * LANES, LANES)
        x_v[r, sl] = x_v[r, sl] + rows_v[r, sl]
        return 0
      return lax.fori_loop(0, D_MODEL // LANES, add_vec, 0)

    lax.fori_loop(0, CHUNK, add_row, 0)
    pltpu.sync_copy(x_v, out_hbm.at[pl.ds(r0, CHUNK)])
    return 0

  lax.fori_loop(0, NCHUNK, chunk_body, 0)


@functools.partial(jax.jit, donate_argnums=())
def _lookup_add(x2, ids, table):
  mesh = plsc.VectorSubcoreMesh(
      core_axis_name="c", subcore_axis_name="s",
      num_cores=NUM_CORES, num_subcores=NUM_SUBCORES,
  )
  fn = pl.kernel(
      _sc_body,
      out_type=jax.ShapeDtypeStruct((ROWS, D_MODEL), jnp.float32),
      mesh=mesh,
      compiler_params=pltpu.CompilerParams(needs_layout_passes=False),
      scratch_types=[
          pltpu.VMEM((SEQ,), jnp.int32),
          pltpu.VMEM((RPW,), jnp.int32),
          pltpu.VMEM((CHUNK, D_MODEL), jnp.float32),
          pltpu.VMEM((CHUNK, D_MODEL), jnp.float32),
          pltpu.SemaphoreType.DMA,
      ],
  )
  return fn(x2, ids, table)


def kernel(X, input_ids, table):
  x2 = X.reshape(ROWS, D_MODEL)
  ids = input_ids.reshape(ROWS)
  out = _lookup_add(x2, ids, table)
  return out.reshape(BATCH, SEQ, D_MODEL)


# speedup vs baseline: 1.6081x; 1.6081x over previous
import functools

import jax
import jax.numpy as jnp
from jax import lax
from jax.experimental import pallas as pl
from jax.experimental.pallas import tpu as pltpu
from jax.experimental.pallas import tpu_sc as plsc

D_MODEL = 1024
BATCH = 4
SEQ = 2048
ROWS = BATCH * SEQ
NUM_CORES = 2
NUM_SUBCORES = 16
NW = NUM_CORES * NUM_SUBCORES
RPW = ROWS // NW
CHUNK = 32
NCHUNK = RPW // CHUNK
LANES = 16
W_PER_BATCH = NW // BATCH


def _sc_body(x_hbm, ids_hbm, table_hbm, out_hbm, ids_v, idx_v, x_v, rows_v, sem):
  cid = lax.axis_index("c")
  sid = lax.axis_index("s")
  wid = sid * NUM_CORES + cid
  b = wid // W_PER_BATCH
  s_local = (wid % W_PER_BATCH) * RPW

  pltpu.sync_copy(ids_hbm.at[pl.ds(b * SEQ, SEQ)], ids_v)

  def _mask(v):
    return jnp.where(
        v != jnp.zeros((LANES,), jnp.int32),
        jnp.ones((LANES,), jnp.int32),
        jnp.zeros((LANES,), jnp.int32),
    )

  def pre_body(i, carry):
    m = _mask(ids_v[pl.ds(i * LANES, LANES)])
    return carry + jnp.sum(m)

  carry = lax.fori_loop(0, s_local // LANES, pre_body, jnp.int32(0))

  def pos_body(j, carry):
    m = _mask(ids_v[pl.ds(s_local + j * LANES, LANES)])
    pos = (plsc.cumsum(m) + carry) * m
    idx_v[pl.ds(j * LANES, LANES)] = pos
    return carry + jnp.sum(m)

  lax.fori_loop(0, RPW // LANES, pos_body, carry)

  def chunk_body(c, _):
    r0 = wid * RPW + c * CHUNK
    gather = pltpu.async_copy(
        table_hbm.at[idx_v.at[pl.ds(c * CHUNK, CHUNK)]], rows_v, sem
    )
    pltpu.sync_copy(x_hbm.at[pl.ds(r0, CHUNK)], x_v)
    gather.wait()

    def add_row(r, _):
      for v in range(D_MODEL // LANES):
        sl = pl.ds(v * LANES, LANES)
        plsc.addupdate(x_v.at[r, sl], rows_v[r, sl])
      return 0

    lax.fori_loop(0, CHUNK, add_row, 0)
    pltpu.sync_copy(x_v, out_hbm.at[pl.ds(r0, CHUNK)])
    return 0

  lax.fori_loop(0, NCHUNK, chunk_body, 0)


@functools.partial(jax.jit, donate_argnums=())
def _lookup_add(x2, ids, table):
  mesh = plsc.VectorSubcoreMesh(
      core_axis_name="c", subcore_axis_name="s",
      num_cores=NUM_CORES, num_subcores=NUM_SUBCORES,
  )
  fn = pl.kernel(
      _sc_body,
      out_type=jax.ShapeDtypeStruct((ROWS, D_MODEL), jnp.float32),
      mesh=mesh,
      compiler_params=pltpu.CompilerParams(needs_layout_passes=False),
      scratch_types=[
          pltpu.VMEM((SEQ,), jnp.int32),
          pltpu.VMEM((RPW,), jnp.int32),
          pltpu.VMEM((CHUNK, D_MODEL), jnp.float32),
          pltpu.VMEM((CHUNK, D_MODEL), jnp.float32),
          pltpu.SemaphoreType.DMA,
      ],
  )
  return fn(x2, ids, table)


def kernel(X, input_ids, table):
  x2 = X.reshape(ROWS, D_MODEL)
  ids = input_ids.reshape(ROWS)
  out = _lookup_add(x2, ids, table)
  return out.reshape(BATCH, SEQ, D_MODEL)


# speedup vs baseline: 2.1478x; 1.3356x over previous
import functools

import jax
import jax.numpy as jnp
from jax import lax
from jax.experimental import pallas as pl
from jax.experimental.pallas import tpu as pltpu
from jax.experimental.pallas import tpu_sc as plsc

D_MODEL = 1024
BATCH = 4
SEQ = 2048
ROWS = BATCH * SEQ
NUM_CORES = 2
NUM_SUBCORES = 16
NW = NUM_CORES * NUM_SUBCORES
RPW = ROWS // NW
CHUNK = 16
NCHUNK = RPW // CHUNK
LANES = 16
W_PER_BATCH = NW // BATCH


def _sc_body(x_hbm, ids_hbm, table_hbm, out_hbm,
             ids_v, idx_v, x0, x1, r0b, r1b, sd0, sd1, so0, so1):
  cid = lax.axis_index("c")
  sid = lax.axis_index("s")
  wid = sid * NUM_CORES + cid
  b = wid // W_PER_BATCH
  s_local = (wid % W_PER_BATCH) * RPW
  row0 = wid * RPW

  pltpu.async_copy(x_hbm.at[pl.ds(row0, CHUNK)], x0, sd0)

  pltpu.sync_copy(ids_hbm.at[pl.ds(b * SEQ, SEQ)], ids_v)

  def _mask(v):
    return jnp.where(
        v != jnp.zeros((LANES,), jnp.int32),
        jnp.ones((LANES,), jnp.int32),
        jnp.zeros((LANES,), jnp.int32),
    )

  def pre_body(i, carry):
    m = _mask(ids_v[pl.ds(i * LANES, LANES)])
    return carry + jnp.sum(m)

  carry = lax.fori_loop(0, s_local // LANES, pre_body, jnp.int32(0))

  def pos_body(j, carry):
    m = _mask(ids_v[pl.ds(s_local + j * LANES, LANES)])
    pos = (plsc.cumsum(m) + carry) * m
    idx_v[pl.ds(j * LANES, LANES)] = pos
    return carry + jnp.sum(m)

  lax.fori_loop(0, RPW // LANES, pos_body, carry)

  def issue_gather(c, rbuf, sem):
    pltpu.async_copy(table_hbm.at[idx_v.at[pl.ds(c * CHUNK, CHUNK)]], rbuf, sem)

  def issue_x(c, xbuf, sem):
    pltpu.async_copy(x_hbm.at[pl.ds(row0 + c * CHUNK, CHUNK)], xbuf, sem)

  def wait_in(xbuf, rbuf, sem):
    pltpu.make_async_copy(x_hbm.at[pl.ds(0, CHUNK)], xbuf, sem).wait()
    pltpu.make_async_copy(x_hbm.at[pl.ds(0, CHUNK)], rbuf, sem).wait()

  def start_store(c, xbuf, sem):
    pltpu.async_copy(xbuf, out_hbm.at[pl.ds(row0 + c * CHUNK, CHUNK)], sem)

  def wait_store(xbuf, sem):
    pltpu.make_async_copy(xbuf, out_hbm.at[pl.ds(0, CHUNK)], sem).wait()

  def add_chunk(xbuf, rbuf):
    def add_row(r, _):
      for v in range(D_MODEL // LANES):
        sl = pl.ds(v * LANES, LANES)
        plsc.addupdate(xbuf.at[r, sl], rbuf[r, sl])
      return 0
    lax.fori_loop(0, CHUNK, add_row, 0)

  issue_gather(0, r0b, sd0)
  issue_x(1, x1, sd1)
  issue_gather(1, r1b, sd1)
  wait_in(x0, r0b, sd0)
  add_chunk(x0, r0b)
  start_store(0, x0, so0)

  def pair(i, _):
    c = 1 + 2 * i
    wait_store(x0, so0)
    issue_x(c + 1, x0, sd0)
    issue_gather(c + 1, r0b, sd0)
    wait_in(x1, r1b, sd1)
    add_chunk(x1, r1b)
    start_store(c, x1, so1)
    wait_store(x1, so1)
    issue_x(c + 2, x1, sd1)
    issue_gather(c + 2, r1b, sd1)
    wait_in(x0, r0b, sd0)
    add_chunk(x0, r0b)
    start_store(c + 1, x0, so0)
    return 0

  lax.fori_loop(0, (NCHUNK - 2) // 2, pair, 0)

  wait_store(x0, so0)
  wait_in(x1, r1b, sd1)
  add_chunk(x1, r1b)
  start_store(NCHUNK - 1, x1, so1)
  wait_store(x1, so1)


@functools.partial(jax.jit, donate_argnums=())
def _lookup_add(x2, ids, table):
  mesh = plsc.VectorSubcoreMesh(
      core_axis_name="c", subcore_axis_name="s",
      num_cores=NUM_CORES, num_subcores=NUM_SUBCORES,
  )
  fn = pl.kernel(
      _sc_body,
      out_type=jax.ShapeDtypeStruct((ROWS, D_MODEL), jnp.float32),
      mesh=mesh,
      compiler_params=pltpu.CompilerParams(needs_layout_passes=False),
      scratch_types=[
          pltpu.VMEM((SEQ,), jnp.int32),
          pltpu.VMEM((RPW,), jnp.int32),
          pltpu.VMEM((CHUNK, D_MODEL), jnp.float32),
          pltpu.VMEM((CHUNK, D_MODEL), jnp.float32),
          pltpu.VMEM((CHUNK, D_MODEL), jnp.float32),
          pltpu.VMEM((CHUNK, D_MODEL), jnp.float32),
          pltpu.SemaphoreType.DMA,
          pltpu.SemaphoreType.DMA,
          pltpu.SemaphoreType.DMA,
          pltpu.SemaphoreType.DMA,
      ],
  )
  return fn(x2, ids, table)


def kernel(X, input_ids, table):
  x2 = X.reshape(ROWS, D_MODEL)
  ids = input_ids.reshape(ROWS)
  out = _lookup_add(x2, ids, table)
  return out.reshape(BATCH, SEQ, D_MODEL)


# speedup vs baseline: 2.1750x; 1.0127x over previous
import functools

import jax
import jax.numpy as jnp
from jax import lax
from jax.experimental import pallas as pl
from jax.experimental.pallas import tpu as pltpu
from jax.experimental.pallas import tpu_sc as plsc

D_MODEL = 1024
BATCH = 4
SEQ = 2048
ROWS = BATCH * SEQ
NUM_CORES = 2
NUM_SUBCORES = 16
NW = NUM_CORES * NUM_SUBCORES
RPW = ROWS // NW
CHUNK = 16
NCHUNK = RPW // CHUNK
LANES = 16
W_PER_BATCH = NW // BATCH


def _sc_body(x_hbm, ids_hbm, table_hbm, out_hbm,
             ids_v, idx_v, x0, x1, x2, r0b, r1b, r2b,
             sd0, sd1, sd2, so0, so1, so2):
  xb = (x0, x1, x2)
  rb = (r0b, r1b, r2b)
  sd = (sd0, sd1, sd2)
  so = (so0, so1, so2)
  cid = lax.axis_index("c")
  sid = lax.axis_index("s")
  wid = sid * NUM_CORES + cid
  b = wid // W_PER_BATCH
  s_local = (wid % W_PER_BATCH) * RPW
  row0 = wid * RPW

  pltpu.async_copy(x_hbm.at[pl.ds(row0, CHUNK)], x0, sd0)

  pltpu.sync_copy(ids_hbm.at[pl.ds(b * SEQ, SEQ)], ids_v)

  def _mask(v):
    return jnp.where(
        v != jnp.zeros((LANES,), jnp.int32),
        jnp.ones((LANES,), jnp.int32),
        jnp.zeros((LANES,), jnp.int32),
    )

  def pre_body(i, carry):
    m = _mask(ids_v[pl.ds(i * LANES, LANES)])
    return carry + jnp.sum(m)

  carry = lax.fori_loop(0, s_local // LANES, pre_body, jnp.int32(0))

  def pos_body(j, carry):
    m = _mask(ids_v[pl.ds(s_local + j * LANES, LANES)])
    pos = (plsc.cumsum(m) + carry) * m
    idx_v[pl.ds(j * LANES, LANES)] = pos
    return carry + jnp.sum(m)

  lax.fori_loop(0, RPW // LANES, pos_body, carry)

  def issue_in(c, s):
    pltpu.async_copy(x_hbm.at[pl.ds(row0 + c * CHUNK, CHUNK)], xb[s], sd[s])
    pltpu.async_copy(
        table_hbm.at[idx_v.at[pl.ds(c * CHUNK, CHUNK)]], rb[s], sd[s])

  def wait_in(s):
    pltpu.make_async_copy(x_hbm.at[pl.ds(0, CHUNK)], xb[s], sd[s]).wait()
    pltpu.make_async_copy(x_hbm.at[pl.ds(0, CHUNK)], rb[s], sd[s]).wait()

  def start_store(c, s):
    pltpu.async_copy(xb[s], out_hbm.at[pl.ds(row0 + c * CHUNK, CHUNK)], so[s])

  def wait_store(s):
    pltpu.make_async_copy(xb[s], out_hbm.at[pl.ds(0, CHUNK)], so[s]).wait()

  def add_chunk(s):
    def add_row(r, _):
      for v in range(D_MODEL // LANES):
        sl = pl.ds(v * LANES, LANES)
        plsc.addupdate(xb[s].at[r, sl], rb[s][r, sl])
      return 0
    lax.fori_loop(0, CHUNK, add_row, 0)

  def body(c, s, first=False, issue_next=True):
    wait_in(s)
    add_chunk(s)
    if not first:
      wait_store((s + 2) % 3)
    if issue_next:
      issue_in(c + 2, (s + 2) % 3)
    start_store(c, s)

  pltpu.async_copy(
      table_hbm.at[idx_v.at[pl.ds(0, CHUNK)]], rb[0], sd[0])
  issue_in(1, 1)

  body(0, 0, first=True)
  def triple(i, _):
    c = 1 + 3 * i
    body(c, 1)
    body(c + 1, 2)
    body(c + 2, 0)
    return 0

  lax.fori_loop(0, (NCHUNK - 4) // 3, triple, 0)

  body(NCHUNK - 3, 1)
  body(NCHUNK - 2, 2, issue_next=False)
  body(NCHUNK - 1, 0, issue_next=False)
  wait_store((NCHUNK - 1) % 3)


@functools.partial(jax.jit, donate_argnums=())
def _lookup_add(x2, ids, table):
  mesh = plsc.VectorSubcoreMesh(
      core_axis_name="c", subcore_axis_name="s",
      num_cores=NUM_CORES, num_subcores=NUM_SUBCORES,
  )
  fn = pl.kernel(
      _sc_body,
      out_type=jax.ShapeDtypeStruct((ROWS, D_MODEL), jnp.float32),
      mesh=mesh,
      compiler_params=pltpu.CompilerParams(needs_layout_passes=False),
      scratch_types=[
          pltpu.VMEM((SEQ,), jnp.int32),
          pltpu.VMEM((RPW,), jnp.int32),
          pltpu.VMEM((CHUNK, D_MODEL), jnp.float32),
          pltpu.VMEM((CHUNK, D_MODEL), jnp.float32),
          pltpu.VMEM((CHUNK, D_MODEL), jnp.float32),
          pltpu.VMEM((CHUNK, D_MODEL), jnp.float32),
          pltpu.VMEM((CHUNK, D_MODEL), jnp.float32),
          pltpu.VMEM((CHUNK, D_MODEL), jnp.float32),
          pltpu.SemaphoreType.DMA,
          pltpu.SemaphoreType.DMA,
          pltpu.SemaphoreType.DMA,
          pltpu.SemaphoreType.DMA,
          pltpu.SemaphoreType.DMA,
          pltpu.SemaphoreType.DMA,
      ],
  )
  return fn(x2, ids, table)


def kernel(X, input_ids, table):
  x2 = X.reshape(ROWS, D_MODEL)
  ids = input_ids.reshape(ROWS)
  out = _lookup_add(x2, ids, table)
  return out.reshape(BATCH, SEQ, D_MODEL)
